# baseline (device time: 18369 ns/iter reference)
import jax
import jax.numpy as jnp
from jax import lax
from jax.experimental import pallas as pl
from jax.experimental.pallas import tpu as pltpu

N_DEV = 8


def kernel(A, B):
    m, k = A.shape
    k2, n = B.shape
    mc = m // N_DEV

    def body(a_ref, b_ref, out_ref, part_ref, p1_ref, g_ref,
             send_p1, recv_p1, send_p2, recv_p2):
        my = lax.axis_index("i")

        barrier_sem = pltpu.get_barrier_semaphore()
        for d in range(1, N_DEV):
            pl.semaphore_signal(
                barrier_sem, inc=1,
                device_id=((my + d) % N_DEV,),
                device_id_type=pl.DeviceIdType.MESH,
            )

        b_bf = b_ref[:, :].astype(jnp.bfloat16)

        p1_sends = []
        for d in range(1, N_DEV):
            tgt = (my + d) % N_DEV
            part_ref[pl.ds(tgt, 1), :, :] = jnp.dot(
                a_ref[pl.ds(tgt * mc, mc), :].astype(jnp.bfloat16),
                b_bf,
                preferred_element_type=jnp.float32,
            ).astype(jnp.bfloat16)[None]
            if d == 1:
                pl.semaphore_wait(barrier_sem, N_DEV - 1)
            rdma = pltpu.make_async_remote_copy(
                src_ref=part_ref.at[tgt],
                dst_ref=p1_ref.at[my],
                send_sem=send_p1.at[d - 1],
                recv_sem=recv_p1.at[my],
                device_id=(tgt,),
                device_id_type=pl.DeviceIdType.MESH,
            )
            rdma.start()
            p1_sends.append(rdma)

        z = jnp.dot(
            a_ref[pl.ds(my * mc, mc), :].astype(jnp.bfloat16),
            b_bf,
            preferred_element_type=jnp.float32,
        )

        for d in range(1, N_DEV):
            src = (my + d) % N_DEV
            recv = pltpu.make_async_remote_copy(
                src_ref=p1_ref.at[src],
                dst_ref=p1_ref.at[src],
                send_sem=send_p1.at[d - 1],
                recv_sem=recv_p1.at[src],
                device_id=(src,),
                device_id_type=pl.DeviceIdType.MESH,
            )
            recv.wait_recv()
            z += p1_ref[pl.ds(src, 1), :, :].astype(jnp.float32)[0]

        silu = z / (1.0 + jnp.exp(-z))
        g_ref[pl.ds(my, 1), :, :] = silu.astype(jnp.bfloat16)[None]

        p2_sends = []
        for d in range(1, N_DEV):
            tgt = (my + d) % N_DEV
            rdma = pltpu.make_async_remote_copy(
                src_ref=g_ref.at[my],
                dst_ref=g_ref.at[my],
                send_sem=send_p2.at[d - 1],
                recv_sem=recv_p2.at[my],
                device_id=(tgt,),
                device_id_type=pl.DeviceIdType.MESH,
            )
            rdma.start()
            p2_sends.append(rdma)

        out_ref[pl.ds(my * mc, mc), :] = silu

        for rdma in p1_sends:
            rdma.wait_send()

        for d in range(1, N_DEV):
            src = (my + d) % N_DEV
            recv = pltpu.make_async_remote_copy(
                src_ref=g_ref.at[src],
                dst_ref=g_ref.at[src],
                send_sem=send_p2.at[d - 1],
                recv_sem=recv_p2.at[src],
                device_id=(src,),
                device_id_type=pl.DeviceIdType.MESH,
            )
            recv.wait_recv()
            out_ref[pl.ds(src * mc, mc), :] = (
                g_ref[pl.ds(src, 1), :, :].astype(jnp.float32)[0]
            )

        for rdma in p2_sends:
            rdma.wait_send()

    return pl.pallas_call(
        body,
        out_shape=jax.ShapeDtypeStruct((m, n), jnp.float32),
        in_specs=[
            pl.BlockSpec(memory_space=pltpu.VMEM),
            pl.BlockSpec(memory_space=pltpu.VMEM),
        ],
        out_specs=pl.BlockSpec(memory_space=pltpu.VMEM),
        scratch_shapes=[
            pltpu.VMEM((N_DEV, m // N_DEV, n), jnp.bfloat16),
            pltpu.VMEM((N_DEV, m // N_DEV, n), jnp.bfloat16),
            pltpu.VMEM((N_DEV, m // N_DEV, n), jnp.bfloat16),
            pltpu.SemaphoreType.DMA((N_DEV - 1,)),
            pltpu.SemaphoreType.DMA((N_DEV,)),
            pltpu.SemaphoreType.DMA((N_DEV - 1,)),
            pltpu.SemaphoreType.DMA((N_DEV,)),
        ],
        compiler_params=pltpu.CompilerParams(collective_id=0),
    )(A, B)
